# baseline (speedup 1.0000x reference)
import functools

import jax
import jax.numpy as jnp
from jax import lax
from jax.experimental import pallas as pl
from jax.experimental.pallas import tpu as pltpu
from jax.experimental.pallas import tpu_sc as plsc

WORD_DIM = 64
INPUT_DIM = 128
BATCH = 16384
TROWS = 250000

NC = 2
NS = 16
NW = NC * NS
B_PER_W = BATCH // NW
CHUNK = 128
N_CHUNKS = B_PER_W // CHUNK

_sc_mesh = plsc.VectorSubcoreMesh(core_axis_name="c", subcore_axis_name="s")


@functools.partial(
    pl.kernel,
    mesh=_sc_mesh,
    out_type=jax.ShapeDtypeStruct((BATCH, INPUT_DIM), jnp.int32),
    scratch_types=[
        pltpu.VMEM((N_CHUNKS, CHUNK), jnp.int32),
        pltpu.VMEM((2, CHUNK, INPUT_DIM), jnp.int32),
        pltpu.SemaphoreType.DMA,
        pltpu.SemaphoreType.DMA,
    ],
)
def _gather_sc(tidx_hbm, table_hbm, out_hbm, idx_v, rows_v, sem0, sem1):
    wid = lax.axis_index("s") * NC + lax.axis_index("c")
    base = wid * B_PER_W
    sems = (sem0, sem1)
    pltpu.sync_copy(tidx_hbm.at[pl.ds(wid * N_CHUNKS, N_CHUNKS)], idx_v)
    cp = pltpu.async_copy(table_hbm.at[idx_v.at[0]], rows_v.at[0], sem0)
    for j in range(N_CHUNKS):
        nxt = None
        if j + 1 < N_CHUNKS:
            nxt = pltpu.async_copy(
                table_hbm.at[idx_v.at[j + 1]], rows_v.at[(j + 1) % 2],
                sems[(j + 1) % 2])
        cp.wait()
        pltpu.sync_copy(rows_v.at[j % 2],
                        out_hbm.at[pl.ds(base + j * CHUNK, CHUNK)])
        cp = nxt


_BLK = 2048


def _proj_body(rp_ref, rl_ref, g_ref, w_ref, b_ref, o_ref):
    g = g_ref[...]
    rp = rp_ref[...]
    rl = rl_ref[...]
    w_sel = jnp.where(rp == 0, g[:, :WORD_DIM], g[:, WORD_DIM:])
    bits = jnp.where(
        rl == 0,
        lax.shift_left(w_sel, 16),
        lax.bitwise_and(w_sel, jnp.int32(-65536)),
    )
    x = lax.bitcast_convert_type(bits, jnp.float32)
    acc = jnp.dot(x, w_ref[...], preferred_element_type=jnp.float32)
    o_ref[...] = jnp.tanh(acc + b_ref[...])


def _proj_tc(rp, rl, g, W, b):
    return pl.pallas_call(
        _proj_body,
        grid=(BATCH // _BLK,),
        in_specs=[
            pl.BlockSpec((_BLK, 1), lambda i: (i, 0)),
            pl.BlockSpec((_BLK, 1), lambda i: (i, 0)),
            pl.BlockSpec((_BLK, INPUT_DIM), lambda i: (i, 0)),
            pl.BlockSpec((WORD_DIM, INPUT_DIM), lambda i: (0, 0)),
            pl.BlockSpec((1, INPUT_DIM), lambda i: (0, 0)),
        ],
        out_specs=pl.BlockSpec((_BLK, INPUT_DIM), lambda i: (i, 0)),
        out_shape=jax.ShapeDtypeStruct((BATCH, INPUT_DIM), jnp.float32),
    )(rp, rl, g, W, b.reshape(1, INPUT_DIM))


def kernel(word_indices, word_table, W, b):
    idx = word_indices.astype(jnp.int32)
    tidx = (idx // 4).reshape(NW * N_CHUNKS, CHUNK)
    rp = ((idx % 4) // 2).reshape(BATCH, 1)
    rl = (idx % 2).reshape(BATCH, 1)
    tb = word_table.astype(jnp.bfloat16).reshape(TROWS, 2, 2, WORD_DIM)
    tb = tb.transpose(0, 1, 3, 2)
    tbl = lax.bitcast_convert_type(tb, jnp.int32).reshape(TROWS, INPUT_DIM)
    g = _gather_sc(tidx, tbl)
    return _proj_tc(rp, rl, g, W, b)


# speedup vs baseline: 3.0530x; 3.0530x over previous
import functools

import jax
import jax.numpy as jnp
from jax import lax
from jax.experimental import pallas as pl
from jax.experimental.pallas import tpu as pltpu
from jax.experimental.pallas import tpu_sc as plsc

WORD_DIM = 64
INPUT_DIM = 128
BATCH = 16384

NC = 2
NS = 16
NW = NC * NS
B_PER_W = BATCH // NW

_sc_mesh = plsc.VectorSubcoreMesh(core_axis_name="c", subcore_axis_name="s")


@functools.partial(
    pl.kernel,
    mesh=_sc_mesh,
    out_type=jax.ShapeDtypeStruct((BATCH, WORD_DIM), jnp.int32),
    scratch_types=[
        pltpu.VMEM((B_PER_W,), jnp.int32),
        pltpu.VMEM((B_PER_W, WORD_DIM), jnp.int32),
        pltpu.SemaphoreType.DMA,
    ],
)
def _gather_sc(idx_hbm, table_hbm, out_hbm, idx_v, rows_v, sem):
    wid = lax.axis_index("s") * NC + lax.axis_index("c")
    base = wid * B_PER_W
    pltpu.sync_copy(idx_hbm.at[pl.ds(base, B_PER_W)], idx_v)

    def fire_group(g, carry):
        v = idx_v[pl.ds(g * 16, 16)]
        for u in range(16):
            t = v[u]
            pltpu.make_async_copy(
                table_hbm.at[pl.ds(t, 1)],
                rows_v.at[pl.ds(g * 16 + u, 1)],
                sem,
            ).start()
        return carry

    lax.fori_loop(0, B_PER_W // 16, fire_group, 0)
    pltpu.make_async_copy(
        table_hbm.at[pl.ds(0, B_PER_W)], rows_v, sem
    ).wait()
    pltpu.sync_copy(rows_v, out_hbm.at[pl.ds(base, B_PER_W)])


_BLK = 2048


def _proj_body(x_ref, w_ref, b_ref, o_ref):
    x = lax.bitcast_convert_type(x_ref[...], jnp.float32)
    acc = jnp.dot(x, w_ref[...], preferred_element_type=jnp.float32)
    o_ref[...] = jnp.tanh(acc + b_ref[...])


def _proj_tc(x, W, b):
    return pl.pallas_call(
        _proj_body,
        grid=(BATCH // _BLK,),
        in_specs=[
            pl.BlockSpec((_BLK, WORD_DIM), lambda i: (i, 0)),
            pl.BlockSpec((WORD_DIM, INPUT_DIM), lambda i: (0, 0)),
            pl.BlockSpec((1, INPUT_DIM), lambda i: (0, 0)),
        ],
        out_specs=pl.BlockSpec((_BLK, INPUT_DIM), lambda i: (i, 0)),
        out_shape=jax.ShapeDtypeStruct((BATCH, INPUT_DIM), jnp.float32),
    )(x, W, b.reshape(1, INPUT_DIM))


def kernel(word_indices, word_table, W, b):
    idx = word_indices.astype(jnp.int32)
    tbl = lax.bitcast_convert_type(word_table, jnp.int32)
    gathered = _gather_sc(idx, tbl)
    return _proj_tc(gathered, W, b)
